# Initial kernel scaffold; baseline (speedup 1.0000x reference)
import functools

import jax
import jax.numpy as jnp
from jax import lax
from jax.experimental import pallas as pl
from jax.experimental.pallas import tpu as pltpu
from jax.experimental.pallas import tpu_sc as plsc

_N = 10000
_E = 160000
_D = 128
_H = 3
_O = 1024
_NPAD = 10112
_NC = 2
_NS = 16
_NT = _NC * _NS
_EPT = _E // _NT
_RW = 128
_ROWS = 40
_EPTP = _ROWS * _RW


def _prep_body(ft_ref, w3t_ref, al_ref, ar_ref, w1t_ref, woutt_ref,
               bgt_ref, b1c_ref, boutc_ref,
               elt_ref, ert_ref, pt_ref, const_ref):
    mm = (((1,), (0,)), ((), ()))
    ft = ft_ref[...]
    cb = lax.dot_general(woutt_ref[...], b1c_ref[...], mm) + boutc_ref[...]
    for h in range(_H):
        w3t_h = w3t_ref[h]
        al_h = al_ref[pl.ds(h, 1), :]
        ar_h = ar_ref[pl.ds(h, 1), :]
        alt = lax.dot_general(al_h, w3t_h, mm)
        art = lax.dot_general(ar_h, w3t_h, mm)
        w1t_h = w1t_ref[:, pl.ds(h * _O, _O)]
        vht = lax.dot_general(woutt_ref[...], w1t_h, mm)
        qht = lax.dot_general(vht, w3t_h, mm)
        cb = cb + lax.dot_general(vht, bgt_ref[:, pl.ds(h, 1)], mm)
        elt_ref[pl.ds(h, 1), :] = lax.dot_general(alt, ft, mm)
        ert_ref[pl.ds(h, 1), :] = lax.dot_general(art, ft, mm)
        pt_ref[pl.ds(h, 1), :] = lax.dot_general(qht, ft, mm)
    const_ref[...] = cb


def _edge_a_body(elt_hbm, ert_hbm, src_hbm, dst_hbm, zeros_hbm,
                 ee_hbm, pden_hbm,
                 src_v, dst_v, tab_el, tab_er, ee_v, dsh0, dsh1, dsh2):
    cid = lax.axis_index("c")
    sid = lax.axis_index("s")
    lin = cid * _NS + sid
    pltpu.sync_copy(src_hbm.at[lin], src_v)
    pltpu.sync_copy(dst_hbm.at[lin], dst_v)
    dshs = [dsh0, dsh1, dsh2]

    @pl.when(sid == 0)
    def _():
        for h in range(_H):
            pltpu.sync_copy(zeros_hbm, dshs[h])

    plsc.subcore_barrier()
    for h in range(_H):
        pltpu.sync_copy(elt_hbm.at[h], tab_el)
        pltpu.sync_copy(ert_hbm.at[h], tab_er)

        @pl.loop(0, _ROWS)
        def _row(j):
            for k in range(_RW // 16):
                sl = pl.ds(k * 16, 16)
                s16 = src_v[j, sl]
                d16 = dst_v[j, sl]
                e = (plsc.load_gather(tab_el, [s16])
                     + plsc.load_gather(tab_er, [d16]))
                e = jnp.where(e >= 0, e, 0.2 * e)
                ee_v[j, sl] = jnp.exp(e)

        @pl.loop(0, _ROWS)
        def _scat(j):
            pltpu.sync_copy(ee_v.at[j], dshs[h].at[dst_v.at[j]], add=True)

        pltpu.sync_copy(ee_v, ee_hbm.at[h, lin])

    plsc.subcore_barrier()

    @pl.when(sid == 0)
    def _():
        for h in range(_H):
            pltpu.sync_copy(dshs[h], pden_hbm.at[cid, h])


def _edge_b_body(src_hbm, dst_hbm, pden_hbm, pt_hbm, ee_hbm,
                 out_hbm,
                 src_v, dst_v, den_v, den2_v, p_v, ee_v, acc_v):
    cid = lax.axis_index("c")
    sid = lax.axis_index("s")
    lin = cid * _NS + sid
    pltpu.sync_copy(src_hbm.at[lin], src_v)
    pltpu.sync_copy(dst_hbm.at[lin], dst_v)
    for h in range(_H):
        pltpu.sync_copy(pden_hbm.at[0, h], den_v)
        pltpu.sync_copy(pden_hbm.at[1, h], den2_v)

        @pl.loop(0, _NPAD // 16)
        def _sum(i):
            sl = pl.ds(i * 16, 16)
            den_v[sl] = den_v[sl] + den2_v[sl]

        pltpu.sync_copy(pt_hbm.at[h], p_v)
        pltpu.sync_copy(ee_hbm.at[h, lin], ee_v)

        def _row(j, acc):
            for k in range(_RW // 16):
                sl = pl.ds(k * 16, 16)
                d16 = dst_v[j, sl]
                s16 = src_v[j, sl]
                den = plsc.load_gather(den_v, [d16])
                alpha = ee_v[j, sl] / jnp.maximum(den, 1e-9)
                acc = acc + alpha * plsc.load_gather(p_v, [s16])
            return acc

        acc = pl.loop(0, _ROWS, init_carry=jnp.zeros((16,), jnp.float32))(_row)
        acc_v[h, :] = acc
    pltpu.sync_copy(acc_v, out_hbm.at[cid, sid])


def _finish_body(part_ref, const_ref, out_ref):
    tot = jnp.sum(part_ref[...]) * (1.0 / _N) + const_ref[0, 0]
    out_ref[...] = tot * jnp.ones((1, 1), jnp.float32)


def kernel(feats, edge_index, W_gat, attn_l, attn_r, b_gat, W1, b1, Wout, bout):
    f32 = jnp.float32
    featsT = jnp.pad(feats, ((0, _NPAD - _N), (0, 0))).T
    W3T = W_gat.reshape(_D, _H, _O).transpose(1, 2, 0)
    W1T = W1.T
    WoutT = Wout.T
    bgT = b_gat.T
    b1c = b1.reshape(_O, 1)
    boutc = bout.reshape(1, 1)
    src = edge_index[0].reshape(_NT, _EPT)
    dst = edge_index[1].reshape(_NT, _EPT)
    src = jnp.pad(src, ((0, 0), (0, _EPTP - _EPT)),
                  constant_values=_N).reshape(_NT, _ROWS, _RW)
    dst = jnp.pad(dst, ((0, 0), (0, _EPTP - _EPT)),
                  constant_values=_N).reshape(_NT, _ROWS, _RW)
    zeros_n = jnp.zeros((_NPAD,), f32)

    elt, ert, pt, const = pl.pallas_call(
        _prep_body,
        out_shape=(
            jax.ShapeDtypeStruct((_H, _NPAD), f32),
            jax.ShapeDtypeStruct((_H, _NPAD), f32),
            jax.ShapeDtypeStruct((_H, _NPAD), f32),
            jax.ShapeDtypeStruct((1, 1), f32),
        ),
    )(featsT, W3T, attn_l, attn_r, W1T, WoutT, bgT, b1c, boutc)

    mesh = plsc.VectorSubcoreMesh(core_axis_name="c", subcore_axis_name="s",
                                  num_cores=_NC, num_subcores=_NS)

    edge_a = functools.partial(
        pl.kernel,
        out_type=(
            jax.ShapeDtypeStruct((_H, _NT, _ROWS, _RW), f32),
            jax.ShapeDtypeStruct((_NC, _H, _NPAD), f32),
        ),
        mesh=mesh,
        scratch_types=[
            pltpu.VMEM((_ROWS, _RW), jnp.int32),
            pltpu.VMEM((_ROWS, _RW), jnp.int32),
            pltpu.VMEM((_NPAD,), f32),
            pltpu.VMEM((_NPAD,), f32),
            pltpu.VMEM((_ROWS, _RW), f32),
            pltpu.VMEM_SHARED((_NPAD,), f32),
            pltpu.VMEM_SHARED((_NPAD,), f32),
            pltpu.VMEM_SHARED((_NPAD,), f32),
        ],
    )(_edge_a_body)
    ee, pden = edge_a(elt, ert, src, dst, zeros_n)

    edge_b = functools.partial(
        pl.kernel,
        out_type=jax.ShapeDtypeStruct((_NC, _NS, _H, 16), f32),
        mesh=mesh,
        scratch_types=[
            pltpu.VMEM((_ROWS, _RW), jnp.int32),
            pltpu.VMEM((_ROWS, _RW), jnp.int32),
            pltpu.VMEM((_NPAD,), f32),
            pltpu.VMEM((_NPAD,), f32),
            pltpu.VMEM((_NPAD,), f32),
            pltpu.VMEM((_ROWS, _RW), f32),
            pltpu.VMEM((_H, 16), f32),
        ],
    )(_edge_b_body)
    partials = edge_b(src, dst, pden, pt, ee)

    out = pl.pallas_call(
        _finish_body,
        out_shape=jax.ShapeDtypeStruct((1, 1), f32),
    )(partials.reshape(_NT, _H * 16), const)
    return out.astype(jnp.float64)


# speedup vs baseline: 170.2082x; 170.2082x over previous
import functools

import jax
import jax.numpy as jnp
from jax import lax
from jax.experimental import pallas as pl
from jax.experimental.pallas import tpu as pltpu
from jax.experimental.pallas import tpu_sc as plsc

_N = 10000
_E = 160000
_D = 128
_H = 3
_O = 1024
_NPAD = 10112
_NC = 2
_NS = 16
_NT = _NC * _NS
_EPT = _E // _NT
_RW = 128
_ROWS = 40
_EPTP = _ROWS * _RW


def _prep_body(ft_ref, w3t_ref, al_ref, ar_ref, w1t_ref, woutt_ref,
               bgt_ref, b1c_ref, boutc_ref,
               elt_ref, ert_ref, pt_ref, const_ref):
    dn = (((1,), (0,)), ((), ()))
    mm = functools.partial(lax.dot_general, dimension_numbers=dn,
                           precision=lax.Precision.HIGHEST)
    ft = ft_ref[...]
    cb = mm(woutt_ref[...], b1c_ref[...]) + boutc_ref[...]
    for h in range(_H):
        w3t_h = w3t_ref[h]
        al_h = al_ref[pl.ds(h, 1), :]
        ar_h = ar_ref[pl.ds(h, 1), :]
        alt = mm(al_h, w3t_h)
        art = mm(ar_h, w3t_h)
        w1t_h = w1t_ref[:, pl.ds(h * _O, _O)]
        vht = mm(woutt_ref[...], w1t_h)
        qht = mm(vht, w3t_h)
        cb = cb + mm(vht, bgt_ref[:, pl.ds(h, 1)])
        elt_ref[pl.ds(h, 1), :] = mm(alt, ft)
        ert_ref[pl.ds(h, 1), :] = mm(art, ft)
        pt_ref[pl.ds(h, 1), :] = mm(qht, ft)
    const_ref[...] = cb


def _edge_a_body(elt_hbm, ert_hbm, src_hbm, dst_hbm, zeros_hbm,
                 ee_hbm, pden_hbm,
                 src_v, dst_v, tab_el, tab_er, ee_v, dsh0, dsh1, dsh2):
    cid = lax.axis_index("c")
    sid = lax.axis_index("s")
    lin = cid * _NS + sid
    pltpu.sync_copy(src_hbm.at[lin], src_v)
    pltpu.sync_copy(dst_hbm.at[lin], dst_v)
    dshs = [dsh0, dsh1, dsh2]

    @pl.when(sid == 0)
    def _():
        for h in range(_H):
            pltpu.sync_copy(zeros_hbm, dshs[h])

    plsc.subcore_barrier()
    for h in range(_H):
        pltpu.sync_copy(elt_hbm.at[pl.ds(h * _NPAD, _NPAD)], tab_el)
        pltpu.sync_copy(ert_hbm.at[pl.ds(h * _NPAD, _NPAD)], tab_er)

        @pl.loop(0, _ROWS)
        def _row(j):
            for k in range(_RW // 16):
                sl = pl.ds(k * 16, 16)
                s16 = src_v[j, sl]
                d16 = dst_v[j, sl]
                e = (plsc.load_gather(tab_el, [s16])
                     + plsc.load_gather(tab_er, [d16]))
                e = jnp.where(e >= 0, e, 0.2 * e)
                ee_v[j, sl] = jnp.exp(e)

        @pl.loop(0, _ROWS)
        def _scat(j):
            pltpu.sync_copy(ee_v.at[j], dshs[h].at[dst_v.at[j]], add=True)

        pltpu.sync_copy(ee_v, ee_hbm.at[h, lin])

    plsc.subcore_barrier()

    @pl.when(sid == 0)
    def _():
        base = pl.multiple_of(cid * (_H * _NPAD), 128)
        for h in range(_H):
            pltpu.sync_copy(dshs[h], pden_hbm.at[pl.ds(base + h * _NPAD, _NPAD)])


def _edge_b_body(src_hbm, dst_hbm, pden_hbm, pt_hbm, ee_hbm,
                 out_hbm,
                 src_v, dst_v, den_v, den2_v, p_v, ee_v, acc_v):
    cid = lax.axis_index("c")
    sid = lax.axis_index("s")
    lin = cid * _NS + sid
    pltpu.sync_copy(src_hbm.at[lin], src_v)
    pltpu.sync_copy(dst_hbm.at[lin], dst_v)
    for h in range(_H):
        pltpu.sync_copy(pden_hbm.at[pl.ds(h * _NPAD, _NPAD)], den_v)
        pltpu.sync_copy(pden_hbm.at[pl.ds((_H + h) * _NPAD, _NPAD)], den2_v)

        @pl.loop(0, _NPAD // 16)
        def _sum(i):
            sl = pl.ds(i * 16, 16)
            den_v[sl] = den_v[sl] + den2_v[sl]

        pltpu.sync_copy(pt_hbm.at[pl.ds(h * _NPAD, _NPAD)], p_v)
        pltpu.sync_copy(ee_hbm.at[h, lin], ee_v)

        def _row(j, acc):
            for k in range(_RW // 16):
                sl = pl.ds(k * 16, 16)
                d16 = dst_v[j, sl]
                s16 = src_v[j, sl]
                den = plsc.load_gather(den_v, [d16])
                alpha = ee_v[j, sl] / jnp.maximum(den, 1e-9)
                acc = acc + alpha * plsc.load_gather(p_v, [s16])
            return acc

        acc = pl.loop(0, _ROWS, init_carry=jnp.zeros((16,), jnp.float32))(_row)
        acc_v[h, :] = acc
    pltpu.sync_copy(acc_v, out_hbm.at[cid, sid])


def _finish_body(part_ref, const_ref, out_ref):
    tot = jnp.sum(part_ref[...]) * (1.0 / _N) + const_ref[0, 0]
    out_ref[...] = tot * jnp.ones((1, 1), jnp.float32)


def kernel(feats, edge_index, W_gat, attn_l, attn_r, b_gat, W1, b1, Wout, bout):
    f32 = jnp.float32
    featsT = jnp.pad(feats, ((0, _NPAD - _N), (0, 0))).T
    W3T = W_gat.reshape(_D, _H, _O).transpose(1, 2, 0)
    W1T = W1.T
    WoutT = Wout.T
    bgT = b_gat.T
    b1c = b1.reshape(_O, 1)
    boutc = bout.reshape(1, 1)
    src = edge_index[0].reshape(_NT, _EPT)
    dst = edge_index[1].reshape(_NT, _EPT)
    src = jnp.pad(src, ((0, 0), (0, _EPTP - _EPT)),
                  constant_values=_N).reshape(_NT, _ROWS, _RW)
    dst = jnp.pad(dst, ((0, 0), (0, _EPTP - _EPT)),
                  constant_values=_N).reshape(_NT, _ROWS, _RW)
    zeros_n = jnp.zeros((_NPAD,), f32)

    elt, ert, pt, const = pl.pallas_call(
        _prep_body,
        out_shape=(
            jax.ShapeDtypeStruct((_H, _NPAD), f32),
            jax.ShapeDtypeStruct((_H, _NPAD), f32),
            jax.ShapeDtypeStruct((_H, _NPAD), f32),
            jax.ShapeDtypeStruct((1, 1), f32),
        ),
    )(featsT, W3T, attn_l, attn_r, W1T, WoutT, bgT, b1c, boutc)

    mesh = plsc.VectorSubcoreMesh(core_axis_name="c", subcore_axis_name="s",
                                  num_cores=_NC, num_subcores=_NS)

    edge_a = functools.partial(
        pl.kernel,
        out_type=(
            jax.ShapeDtypeStruct((_H, _NT, _ROWS, _RW), f32),
            jax.ShapeDtypeStruct((_NC * _H * _NPAD,), f32),
        ),
        mesh=mesh,
        scratch_types=[
            pltpu.VMEM((_ROWS, _RW), jnp.int32),
            pltpu.VMEM((_ROWS, _RW), jnp.int32),
            pltpu.VMEM((_NPAD,), f32),
            pltpu.VMEM((_NPAD,), f32),
            pltpu.VMEM((_ROWS, _RW), f32),
            pltpu.VMEM_SHARED((_NPAD,), f32),
            pltpu.VMEM_SHARED((_NPAD,), f32),
            pltpu.VMEM_SHARED((_NPAD,), f32),
        ],
        compiler_params=pltpu.CompilerParams(needs_layout_passes=False),
    )(_edge_a_body)
    ee, pden = edge_a(elt.reshape(_H * _NPAD), ert.reshape(_H * _NPAD),
                      src, dst, zeros_n)

    edge_b = functools.partial(
        pl.kernel,
        out_type=jax.ShapeDtypeStruct((_NC, _NS, _H, 16), f32),
        mesh=mesh,
        scratch_types=[
            pltpu.VMEM((_ROWS, _RW), jnp.int32),
            pltpu.VMEM((_ROWS, _RW), jnp.int32),
            pltpu.VMEM((_NPAD,), f32),
            pltpu.VMEM((_NPAD,), f32),
            pltpu.VMEM((_NPAD,), f32),
            pltpu.VMEM((_ROWS, _RW), f32),
            pltpu.VMEM((_H, 16), f32),
        ],
        compiler_params=pltpu.CompilerParams(needs_layout_passes=False),
    )(_edge_b_body)
    partials = edge_b(src, dst, pden, pt.reshape(_H * _NPAD), ee)

    out = pl.pallas_call(
        _finish_body,
        out_shape=jax.ShapeDtypeStruct((1, 1), f32),
    )(partials.reshape(_NT, _H * 16), const)
    return out.astype(jnp.float64)


# speedup vs baseline: 216.7910x; 1.2737x over previous
import functools

import jax
import jax.numpy as jnp
from jax import lax
from jax.experimental import pallas as pl
from jax.experimental.pallas import tpu as pltpu
from jax.experimental.pallas import tpu_sc as plsc

_N = 10000
_E = 160000
_D = 128
_H = 3
_O = 1024
_NPAD = 10112
_NC = 2
_NS = 16
_NT = _NC * _NS
_EPT = _E // _NT
_RW = 128
_ROWS = 40
_EPTP = _ROWS * _RW


def _prep_body(ft_ref, wg_ref, al_ref, ar_ref, w1_ref, woutt_ref,
               bg_ref, b1r_ref, boutc_ref,
               tabs_ref, const_ref):
    dn = (((1,), (1,)), ((), ()))
    mmt = functools.partial(lax.dot_general, dimension_numbers=dn,
                            precision=lax.Precision.HIGHEST)
    v_all = mmt(woutt_ref[...], w1_ref[...])
    cb = jnp.sum(woutt_ref[...] * b1r_ref[...]) + boutc_ref[0, 0]
    rows = []
    for h in range(_H):
        wg_h = wg_ref[:, pl.ds(h * _O, _O)]
        vht = v_all[:, h * _O:(h + 1) * _O]
        lhs = jnp.concatenate(
            [al_ref[pl.ds(h, 1), :], ar_ref[pl.ds(h, 1), :], vht], axis=0)
        rows.append(mmt(lhs, wg_h))
        cb = cb + jnp.sum(vht * bg_ref[pl.ds(h, 1), :])
    tabs_ref[...] = mmt(jnp.concatenate(rows, axis=0), ft_ref[...])
    const_ref[...] = cb * jnp.ones((1, 1), jnp.float32)


def _edge_a_body(tabs_hbm, src_hbm, dst_hbm, zeros_hbm,
                 ee_hbm, pden_hbm,
                 src_v, dst_v, tab_el, tab_er, ee_v, dsh0, dsh1, dsh2):
    cid = lax.axis_index("c")
    sid = lax.axis_index("s")
    lin = cid * _NS + sid
    pltpu.sync_copy(src_hbm.at[lin], src_v)
    pltpu.sync_copy(dst_hbm.at[lin], dst_v)
    dshs = [dsh0, dsh1, dsh2]

    @pl.when(sid == 0)
    def _():
        for h in range(_H):
            pltpu.sync_copy(zeros_hbm, dshs[h])

    plsc.subcore_barrier()
    for h in range(_H):
        pltpu.sync_copy(tabs_hbm.at[pl.ds((3 * h) * _NPAD, _NPAD)], tab_el)
        pltpu.sync_copy(tabs_hbm.at[pl.ds((3 * h + 1) * _NPAD, _NPAD)], tab_er)

        @pl.loop(0, _ROWS)
        def _row(j):
            for k in range(_RW // 16):
                sl = pl.ds(k * 16, 16)
                s16 = src_v[j, sl]
                d16 = dst_v[j, sl]
                e = (plsc.load_gather(tab_el, [s16])
                     + plsc.load_gather(tab_er, [d16]))
                e = jnp.where(e >= 0, e, 0.2 * e)
                ee_v[j, sl] = jnp.exp(e)

        @pl.loop(0, _ROWS)
        def _scat(j):
            pltpu.sync_copy(ee_v.at[j], dshs[h].at[dst_v.at[j]], add=True)

        pltpu.sync_copy(ee_v, ee_hbm.at[h, lin])

    plsc.subcore_barrier()

    @pl.when(sid == 0)
    def _():
        base = pl.multiple_of(cid * (_H * _NPAD), 128)
        for h in range(_H):
            pltpu.sync_copy(dshs[h], pden_hbm.at[pl.ds(base + h * _NPAD, _NPAD)])


def _edge_b_body(src_hbm, dst_hbm, pden_hbm, tabs_hbm, ee_hbm,
                 out_hbm,
                 src_v, dst_v, den_v, den2_v, p_v, ee_v, acc_v):
    cid = lax.axis_index("c")
    sid = lax.axis_index("s")
    lin = cid * _NS + sid
    pltpu.sync_copy(src_hbm.at[lin], src_v)
    pltpu.sync_copy(dst_hbm.at[lin], dst_v)
    for h in range(_H):
        pltpu.sync_copy(pden_hbm.at[pl.ds(h * _NPAD, _NPAD)], den_v)
        pltpu.sync_copy(pden_hbm.at[pl.ds((_H + h) * _NPAD, _NPAD)], den2_v)
        pltpu.sync_copy(tabs_hbm.at[pl.ds((3 * h + 2) * _NPAD, _NPAD)], p_v)
        pltpu.sync_copy(ee_hbm.at[h, lin], ee_v)

        def _row(j, acc):
            for k in range(_RW // 16):
                sl = pl.ds(k * 16, 16)
                d16 = dst_v[j, sl]
                s16 = src_v[j, sl]
                den = (plsc.load_gather(den_v, [d16])
                       + plsc.load_gather(den2_v, [d16]))
                alpha = ee_v[j, sl] / jnp.maximum(den, 1e-9)
                acc = acc + alpha * plsc.load_gather(p_v, [s16])
            return acc

        acc = pl.loop(0, _ROWS, init_carry=jnp.zeros((16,), jnp.float32))(_row)
        acc_v[h, :] = acc
    pltpu.sync_copy(acc_v, out_hbm.at[cid, sid])


def _finish_body(part_ref, const_ref, out_ref):
    tot = jnp.sum(part_ref[...]) * (1.0 / _N) + const_ref[0, 0]
    out_ref[...] = tot * jnp.ones((1, 1), jnp.float32)


def kernel(feats, edge_index, W_gat, attn_l, attn_r, b_gat, W1, b1, Wout, bout):
    f32 = jnp.float32
    featsP = jnp.pad(feats, ((0, _NPAD - _N), (0, 0)))
    WoutT = Wout.T
    b1r = b1.reshape(1, _O)
    boutc = bout.reshape(1, 1)
    src = edge_index[0].reshape(_NT, _EPT)
    dst = edge_index[1].reshape(_NT, _EPT)
    src = jnp.pad(src, ((0, 0), (0, _EPTP - _EPT)),
                  constant_values=_N).reshape(_NT, _ROWS, _RW)
    dst = jnp.pad(dst, ((0, 0), (0, _EPTP - _EPT)),
                  constant_values=_N).reshape(_NT, _ROWS, _RW)
    zeros_n = jnp.zeros((_NPAD,), f32)

    tabs, const = pl.pallas_call(
        _prep_body,
        out_shape=(
            jax.ShapeDtypeStruct((3 * _H, _NPAD), f32),
            jax.ShapeDtypeStruct((1, 1), f32),
        ),
        name="gat_prep",
    )(featsP, W_gat, attn_l, attn_r, W1, WoutT, b_gat, b1r, boutc)
    tabs = tabs.reshape(3 * _H * _NPAD)

    mesh = plsc.VectorSubcoreMesh(core_axis_name="c", subcore_axis_name="s",
                                  num_cores=_NC, num_subcores=_NS)

    edge_a = functools.partial(
        pl.kernel,
        out_type=(
            jax.ShapeDtypeStruct((_H, _NT, _ROWS, _RW), f32),
            jax.ShapeDtypeStruct((_NC * _H * _NPAD,), f32),
        ),
        mesh=mesh,
        scratch_types=[
            pltpu.VMEM((_ROWS, _RW), jnp.int32),
            pltpu.VMEM((_ROWS, _RW), jnp.int32),
            pltpu.VMEM((_NPAD,), f32),
            pltpu.VMEM((_NPAD,), f32),
            pltpu.VMEM((_ROWS, _RW), f32),
            pltpu.VMEM_SHARED((_NPAD,), f32),
            pltpu.VMEM_SHARED((_NPAD,), f32),
            pltpu.VMEM_SHARED((_NPAD,), f32),
        ],
        compiler_params=pltpu.CompilerParams(needs_layout_passes=False),
        name="gat_edge_a",
    )(_edge_a_body)
    ee, pden = edge_a(tabs, src, dst, zeros_n)

    edge_b = functools.partial(
        pl.kernel,
        out_type=jax.ShapeDtypeStruct((_NC, _NS, _H, 16), f32),
        mesh=mesh,
        scratch_types=[
            pltpu.VMEM((_ROWS, _RW), jnp.int32),
            pltpu.VMEM((_ROWS, _RW), jnp.int32),
            pltpu.VMEM((_NPAD,), f32),
            pltpu.VMEM((_NPAD,), f32),
            pltpu.VMEM((_NPAD,), f32),
            pltpu.VMEM((_ROWS, _RW), f32),
            pltpu.VMEM((_H, 16), f32),
        ],
        compiler_params=pltpu.CompilerParams(needs_layout_passes=False),
        name="gat_edge_b",
    )(_edge_b_body)
    partials = edge_b(src, dst, pden, tabs, ee)

    out = pl.pallas_call(
        _finish_body,
        out_shape=jax.ShapeDtypeStruct((1, 1), f32),
        name="gat_finish",
    )(partials.reshape(_NT, _H * 16), const)
    return out.astype(jnp.float64)
